# baseline (speedup 1.0000x reference)
import jax
import jax.numpy as jnp
from jax import lax
from jax.experimental import pallas as pl
from jax.experimental.pallas import tpu as pltpu

_BF = jnp.bfloat16
_F32 = jnp.float32


def _full_spec(a):
    nd = a.ndim
    return pl.BlockSpec(a.shape, lambda b, _nd=nd: (0,) * _nd)


def kernel(image_resnet, params, A, ref_vertices):
    B, D = image_resnet.shape
    n = ref_vertices.shape[0]
    npad = -(-n // 128) * 128

    A_pad = jnp.pad(A, ((0, npad - n), (0, npad - n)))
    refv = jnp.pad(ref_vertices, ((0, npad - n), (0, 0)))
    img3 = image_resnet.reshape(B, 1, D).astype(_BF)

    args = [img3]
    specs = [pl.BlockSpec((1, 1, D), lambda b: (b, 0, 0))]

    def add(a):
        args.append(a)
        specs.append(_full_spec(a))

    add(refv.astype(_BF))
    add(A_pad.astype(_BF))

    W0 = params["lin0"]["W"]
    add(W0[:, :3].T.astype(_BF))
    add(W0[:, 3:].T.astype(_BF))
    add(params["lin0"]["b"].reshape(1, -1))

    blocks = list(params["gc_blocks"]) + list(params["shape_blocks"])
    has_skip = []
    for p in blocks:
        add(p["pre_norm"]["gamma"].reshape(1, -1))
        add(p["pre_norm"]["beta"].reshape(1, -1))
        add(p["lin1"]["W"].T.astype(_BF))
        add(p["lin1"]["b"].reshape(1, -1))
        add(p["norm1"]["gamma"].reshape(1, -1))
        add(p["norm1"]["beta"].reshape(1, -1))
        add(p["conv"]["W"].astype(_BF))
        add(p["conv"]["b"].reshape(1, -1))
        add(p["norm2"]["gamma"].reshape(1, -1))
        add(p["norm2"]["beta"].reshape(1, -1))
        add(p["lin2"]["W"].T.astype(_BF))
        add(p["lin2"]["b"].reshape(1, -1))
        hs = "skip" in p
        has_skip.append(hs)
        if hs:
            add(p["skip"]["W"].T.astype(_BF))
            add(p["skip"]["b"].reshape(1, -1))

    add(params["final_gn"]["gamma"].reshape(1, -1))
    add(params["final_gn"]["beta"].reshape(1, -1))
    add(params["final_lin"]["W"].astype(_BF))
    add(params["final_lin"]["b"].reshape(-1, 1))

    nf = float(n)
    padf = float(npad - n)

    def body(*refs):
        out_ref = refs[-1]
        it = iter(refs[:-1])

        def nxt():
            return next(it)[...]

        def d(u, v):
            return jnp.dot(u, v, preferred_element_type=_F32)

        def dot(a, w):
            return d(a.astype(_BF), w)

        def gn_relu(x, prow, g, bb):
            C = x.shape[1]
            ii = lax.broadcasted_iota(jnp.int32, (C, C), 0) // 8
            jj = lax.broadcasted_iota(jnp.int32, (C, C), 1) // 8
            M = (ii == jj).astype(_BF)
            s = jnp.sum(x, axis=0, keepdims=True) - padf * prow
            s2 = (jnp.sum(x * x, axis=0, keepdims=True)
                  - padf * (prow * prow))
            cnt = 8.0 * nf

            def gsum(v):
                vh = v.astype(_BF)
                vl = (v - vh.astype(_F32)).astype(_BF)
                return d(vh, M) + d(vl, M)

            mean = gsum(s) / cnt
            var = gsum(s2) / cnt - mean * mean
            sc = lax.rsqrt(var + 1e-5) * g
            sh = bb - mean * sc
            return (jnp.maximum(x * sc + sh, 0.0),
                    jnp.maximum(prow * sc + sh, 0.0))

        img = nxt()
        refw = nxt()
        Ab = nxt()
        w3t = nxt()
        wimg = nxt()
        b0 = nxt()

        imgfeat = d(img[0], wimg)
        x = d(refw, w3t) + imgfeat + b0
        prow = imgfeat + b0

        for hs in has_skip:
            gp, bp = nxt(), nxt()
            w1, b1 = nxt(), nxt()
            g1, be1 = nxt(), nxt()
            wc, bc = nxt(), nxt()
            g2, be2 = nxt(), nxt()
            w2, b2 = nxt(), nxt()
            y, py = gn_relu(x, prow, gp, bp)
            y = dot(y, w1) + b1
            py = dot(py, w1) + b1
            y, py = gn_relu(y, py, g1, be1)
            sup = dot(y, wc)
            z = d(Ab, sup.astype(_BF)) + bc
            z, pz = gn_relu(z, bc, g2, be2)
            y2 = dot(z, w2) + b2
            py2 = dot(pz, w2) + b2
            if hs:
                ws, bs = nxt(), nxt()
                xs = dot(x, ws) + bs
                ps = dot(prow, ws) + bs
            else:
                xs = x
                ps = prow
            x = xs + y2
            prow = ps + py2

        gf, bf = nxt(), nxt()
        wf, bfin = nxt(), nxt()
        y, _ = gn_relu(x, prow, gf, bf)
        outT = lax.dot_general(wf, y.astype(_BF), (((1,), (1,)), ((), ())),
                               preferred_element_type=_F32)
        out_ref[0] = (outT + bfin)[:, :n]

    out = pl.pallas_call(
        body,
        grid=(B,),
        in_specs=specs,
        out_specs=pl.BlockSpec((1, 3, n), lambda b: (b, 0, 0)),
        out_shape=jax.ShapeDtypeStruct((B, 3, n), _F32),
        compiler_params=pltpu.CompilerParams(
            dimension_semantics=("parallel",)),
    )(*args)
    return out


# speedup vs baseline: 1.0331x; 1.0331x over previous
import jax
import jax.numpy as jnp
from jax import lax
from jax.experimental import pallas as pl
from jax.experimental.pallas import tpu as pltpu

_BF = jnp.bfloat16
_F32 = jnp.float32


def _full_spec(a):
    nd = a.ndim
    return pl.BlockSpec(a.shape, lambda b, _nd=nd: (0,) * _nd)


def kernel(image_resnet, params, A, ref_vertices):
    B, D = image_resnet.shape
    n = ref_vertices.shape[0]
    npad = -(-n // 128) * 128

    A_pad = jnp.pad(A, ((0, npad - n), (0, npad - n)))
    refv = jnp.pad(ref_vertices, ((0, npad - n), (0, 0)))
    img3 = image_resnet.reshape(B, 1, D).astype(_BF)

    args = [img3]
    specs = [pl.BlockSpec((1, 1, D), lambda b: (b, 0, 0))]

    def add(a):
        args.append(a)
        specs.append(_full_spec(a))

    add(refv.astype(_BF))
    add(A_pad.astype(_BF))

    W0 = params["lin0"]["W"]
    add(W0[:, :3].T.astype(_BF))
    add(W0[:, 3:].T.astype(_BF))
    add(params["lin0"]["b"].reshape(1, -1))

    blocks = list(params["gc_blocks"]) + list(params["shape_blocks"])
    has_skip = []
    for p in blocks:
        add(p["pre_norm"]["gamma"].reshape(1, -1))
        add(p["pre_norm"]["beta"].reshape(1, -1))
        add(p["lin1"]["W"].T.astype(_BF))
        add(p["lin1"]["b"].reshape(1, -1))
        add(p["norm1"]["gamma"].reshape(1, -1))
        add(p["norm1"]["beta"].reshape(1, -1))
        add(p["conv"]["W"].astype(_BF))
        add(p["conv"]["b"].reshape(1, -1))
        add(p["norm2"]["gamma"].reshape(1, -1))
        add(p["norm2"]["beta"].reshape(1, -1))
        add(p["lin2"]["W"].T.astype(_BF))
        add(p["lin2"]["b"].reshape(1, -1))
        hs = "skip" in p
        has_skip.append(hs)
        if hs:
            add(p["skip"]["W"].T.astype(_BF))
            add(p["skip"]["b"].reshape(1, -1))

    add(params["final_gn"]["gamma"].reshape(1, -1))
    add(params["final_gn"]["beta"].reshape(1, -1))
    add(params["final_lin"]["W"].astype(_BF))
    add(params["final_lin"]["b"].reshape(-1, 1))

    nf = float(n)

    def body(*refs):
        out_ref = refs[-1]
        it = iter(refs[:-1])

        def nxt():
            return next(it)[...]

        mask = (lax.broadcasted_iota(jnp.int32, (npad, 1), 0) < n
                ).astype(_F32)

        def d(u, v):
            return jnp.dot(u, v, preferred_element_type=_F32)

        def gn_relu(x, g, bb):
            C = x.shape[1]
            ii = lax.broadcasted_iota(jnp.int32, (C, C), 0) // 8
            jj = lax.broadcasted_iota(jnp.int32, (C, C), 1) // 8
            M = (ii == jj).astype(_BF)
            s = jnp.sum(x, axis=0, keepdims=True)
            s2 = jnp.sum(x * x, axis=0, keepdims=True)
            ss = jnp.concatenate([s, s2], axis=0)
            sh_ = ss.astype(_BF)
            sl_ = (ss - sh_.astype(_F32)).astype(_BF)
            gs = d(sh_, M) + d(sl_, M)
            cnt = 8.0 * nf
            mean = gs[0:1] / cnt
            var = gs[1:2] / cnt - mean * mean
            sc = lax.rsqrt(var + 1e-5) * g
            sh = bb - mean * sc
            return (jnp.maximum(x * sc + sh, 0.0) * mask).astype(_BF)

        img = nxt()
        refw = nxt()
        Ab = nxt()
        w3t = nxt()
        wimg = nxt()
        b0 = nxt()

        x = (d(refw, w3t) + d(img[0], wimg) + b0) * mask

        for hs in has_skip:
            gp, bp = nxt(), nxt()
            w1, b1 = nxt(), nxt()
            g1, be1 = nxt(), nxt()
            wc, bc = nxt(), nxt()
            g2, be2 = nxt(), nxt()
            w2, b2 = nxt(), nxt()
            y = gn_relu(x, gp, bp)
            y1 = (d(y, w1) + b1) * mask
            yg = gn_relu(y1, g1, be1)
            sup = d(yg, wc).astype(_BF)
            z = (d(Ab, sup) + bc) * mask
            zg = gn_relu(z, g2, be2)
            y2 = d(zg, w2) + b2
            if hs:
                ws, bs = nxt(), nxt()
                xs = d(x.astype(_BF), ws) + bs
            else:
                xs = x
            x = (xs + y2) * mask

        gf, bf = nxt(), nxt()
        wf, bfin = nxt(), nxt()
        y = gn_relu(x, gf, bf)
        outT = lax.dot_general(wf, y, (((1,), (1,)), ((), ())),
                               preferred_element_type=_F32)
        out_ref[0] = (outT + bfin)[:, :n]

    out = pl.pallas_call(
        body,
        grid=(B,),
        in_specs=specs,
        out_specs=pl.BlockSpec((1, 3, n), lambda b: (b, 0, 0)),
        out_shape=jax.ShapeDtypeStruct((B, 3, n), _F32),
        compiler_params=pltpu.CompilerParams(
            dimension_semantics=("parallel",)),
    )(*args)
    return out
